# baseline (speedup 1.0000x reference)
import functools

import jax
import jax.numpy as jnp
from jax import lax
from jax.experimental import pallas as pl
from jax.experimental.pallas import tpu as pltpu
from jax.experimental.pallas import tpu_sc as plsc


def _gather_body(table_hbm, idx_hbm, out_hbm, idx_v, rows_v, sem, *,
                 num_cores, b_per_w):
    wid = lax.axis_index("s") * num_cores + lax.axis_index("c")
    base = wid * b_per_w
    pltpu.sync_copy(idx_hbm.at[pl.ds(base, b_per_w)], idx_v)
    pltpu.async_copy(table_hbm.at[idx_v], rows_v, sem).wait()
    pltpu.sync_copy(rows_v, out_hbm.at[pl.ds(base, b_per_w)])


def kernel(kernel, ids):
    table = kernel
    V, D = table.shape
    B = ids.shape[0]
    ids32 = ids.astype(jnp.int32)

    info = plsc.get_sparse_core_info()
    nw = info.num_cores * info.num_subcores
    b_per_w = B // nw

    mesh = plsc.VectorSubcoreMesh(core_axis_name="c", subcore_axis_name="s")
    body = functools.partial(_gather_body, num_cores=info.num_cores,
                             b_per_w=b_per_w)
    run = pl.kernel(
        body,
        mesh=mesh,
        out_type=jax.ShapeDtypeStruct((B, D), jnp.float32),
        scratch_types=[
            pltpu.VMEM((b_per_w,), jnp.int32),
            pltpu.VMEM((b_per_w, D), jnp.float32),
            pltpu.SemaphoreType.DMA,
        ],
        compiler_params=pltpu.CompilerParams(use_tc_tiling_on_sc=False),
    )
    return run(table, ids32)


# speedup vs baseline: 1.7712x; 1.7712x over previous
import functools

import jax
import jax.numpy as jnp
from jax import lax
from jax.experimental import pallas as pl
from jax.experimental.pallas import tpu as pltpu
from jax.experimental.pallas import tpu_sc as plsc


def _gather_body(tableT_hbm, ids_hbm, outT_hbm, row_v, ids_v, orow_v, sem,
                 *, num_cores, rows_per_w, ic, n_ic):
    wid = lax.axis_index("s") * num_cores + lax.axis_index("c")
    for p in range(rows_per_w):
        row = wid * rows_per_w + p
        pltpu.async_copy(tableT_hbm.at[row], row_v, sem).wait()
        for c in range(n_ic):
            pltpu.sync_copy(ids_hbm.at[pl.ds(c * ic, ic)], ids_v)

            def gather_iter(j, _):
                idx = ids_v[pl.ds(j * 16, 16)]
                orow_v[pl.ds(j * 16, 16)] = plsc.load_gather(row_v, [idx])
                return 0

            lax.fori_loop(0, ic // 16, gather_iter, 0, unroll=8)
            pltpu.sync_copy(orow_v, outT_hbm.at[row, pl.ds(c * ic, ic)])


def kernel(kernel, ids):
    table = kernel
    V, D = table.shape
    B = ids.shape[0]
    ids32 = ids.astype(jnp.int32)
    tableT = table.T

    info = plsc.get_sparse_core_info()
    nw = info.num_cores * info.num_subcores
    rows_per_w = D // nw
    ic = 8192
    n_ic = B // ic

    mesh = plsc.VectorSubcoreMesh(core_axis_name="c", subcore_axis_name="s")
    body = functools.partial(_gather_body, num_cores=info.num_cores,
                             rows_per_w=rows_per_w, ic=ic, n_ic=n_ic)
    run = pl.kernel(
        body,
        mesh=mesh,
        out_type=jax.ShapeDtypeStruct((D, B), jnp.float32),
        scratch_types=[
            pltpu.VMEM((V,), jnp.float32),
            pltpu.VMEM((ic,), jnp.int32),
            pltpu.VMEM((ic,), jnp.float32),
            pltpu.SemaphoreType.DMA,
        ],
        compiler_params=pltpu.CompilerParams(needs_layout_passes=False),
    )
    outT = run(tableT, ids32)
    return outT.T


# speedup vs baseline: 2.7407x; 1.5474x over previous
import functools

import jax
import jax.numpy as jnp
from jax import lax
from jax.experimental import pallas as pl
from jax.experimental.pallas import tpu as pltpu
from jax.experimental.pallas import tpu_sc as plsc


def _gather_body(tableT_hbm, ids_hbm, outT_hbm, row_v, ids_v, o0_v, o1_v,
                 sem_r, sem_i, sem_o0, sem_o1, *,
                 num_cores, rows_per_w, oc, n_oc):
    wid = lax.axis_index("s") * num_cores + lax.axis_index("c")
    obufs = (o0_v, o1_v)
    osems = (sem_o0, sem_o1)
    pending = [None, None]

    row_cp = pltpu.async_copy(tableT_hbm.at[wid * rows_per_w], row_v, sem_r)
    pltpu.async_copy(ids_hbm, ids_v, sem_i).wait()
    row_cp.wait()

    for p in range(rows_per_w):
        row = wid * rows_per_w + p
        for c in range(n_oc):
            ob = obufs[c % 2]
            if pending[c % 2] is not None:
                pending[c % 2].wait()
                pending[c % 2] = None
            base = c * oc

            @plsc.parallel_loop(0, oc // 16, 1, unroll=8)
            def gather_iter(j):
                idx = ids_v[pl.ds(base + j * 16, 16)]
                ob[pl.ds(j * 16, 16)] = plsc.load_gather(row_v, [idx])

            if c == n_oc - 1 and p + 1 < rows_per_w:
                row_cp = pltpu.async_copy(
                    tableT_hbm.at[row + 1], row_v, sem_r)
            pending[c % 2] = pltpu.async_copy(
                ob, outT_hbm.at[row, pl.ds(base, oc)], osems[c % 2])
        if p + 1 < rows_per_w:
            row_cp.wait()
    for q in range(2):
        if pending[q] is not None:
            pending[q].wait()


def kernel(kernel, ids):
    table = kernel
    V, D = table.shape
    B = ids.shape[0]
    ids32 = ids.astype(jnp.int32)
    tableT = table.T

    info = plsc.get_sparse_core_info()
    nw = info.num_cores * info.num_subcores
    rows_per_w = D // nw
    oc = 4096
    n_oc = B // oc

    mesh = plsc.VectorSubcoreMesh(core_axis_name="c", subcore_axis_name="s")
    body = functools.partial(_gather_body, num_cores=info.num_cores,
                             rows_per_w=rows_per_w, oc=oc, n_oc=n_oc)
    run = pl.kernel(
        body,
        mesh=mesh,
        out_type=jax.ShapeDtypeStruct((D, B), jnp.float32),
        scratch_types=[
            pltpu.VMEM((V,), jnp.float32),
            pltpu.VMEM((B,), jnp.int32),
            pltpu.VMEM((oc,), jnp.float32),
            pltpu.VMEM((oc,), jnp.float32),
            pltpu.SemaphoreType.DMA,
            pltpu.SemaphoreType.DMA,
            pltpu.SemaphoreType.DMA,
            pltpu.SemaphoreType.DMA,
        ],
        compiler_params=pltpu.CompilerParams(needs_layout_passes=False),
    )
    outT = run(tableT, ids32)
    return outT.T
